# baseline (speedup 1.0000x reference)
import jax
import jax.numpy as jnp
from jax.experimental import pallas as pl
from jax.experimental.pallas import tpu as pltpu

_STREAMS = 4
_ROW_TILE = 1600


def _fused_heads_kernel(x0_ref, x1_ref, x2_ref, x3_ref, w_ref, b_ref, o_ref):
    w = w_ref[...]
    b = b_ref[...]
    t = _ROW_TILE
    wb = w.astype(jnp.bfloat16)
    for j, x_ref in enumerate((x0_ref, x1_ref, x2_ref, x3_ref)):
        o_ref[pl.ds(j * t, t), :] = (
            jnp.dot(
                x_ref[...].astype(jnp.bfloat16),
                wb,
                preferred_element_type=jnp.float32,
            )
            + b
        )


def kernel(local_features, W_coords, b_coords, W_pres, b_pres):
    B, C, R, D = local_features.shape
    M = B * C * R
    x = local_features.reshape(M, D)
    w = jnp.concatenate([W_coords, W_pres], axis=0).T
    b = jnp.concatenate([b_coords, b_pres], axis=0).reshape(1, 5)

    S, tile = _STREAMS, _ROW_TILE
    grid = (M // (S * tile),)

    def x_map(j):
        return lambda i: (S * i + j, 0)

    out = pl.pallas_call(
        _fused_heads_kernel,
        grid=grid,
        in_specs=[pl.BlockSpec((tile, D), x_map(j)) for j in range(S)]
        + [
            pl.BlockSpec((D, 5), lambda i: (0, 0)),
            pl.BlockSpec((1, 5), lambda i: (0, 0)),
        ],
        out_specs=pl.BlockSpec((S * tile, 5), lambda i: (i, 0)),
        out_shape=jax.ShapeDtypeStruct((M, 5), jnp.float32),
        compiler_params=pltpu.CompilerParams(
            dimension_semantics=("arbitrary",),
        ),
    )(x, x, x, x, w, b)

    out = out.reshape(B, C, R, 5)
    return (out[..., :4], out[..., 4:])


# speedup vs baseline: 1.0099x; 1.0099x over previous
import jax
import jax.numpy as jnp
from jax.experimental import pallas as pl
from jax.experimental.pallas import tpu as pltpu

_TILE = 1600
_NBUF = 6


def _fused_heads_kernel(x_hbm, w_ref, b_ref, o_hbm, *scratch):
    xbufs = scratch[:_NBUF]
    obufs = scratch[_NBUF : 2 * _NBUF]
    xsems = scratch[2 * _NBUF : 3 * _NBUF]
    osems = scratch[3 * _NBUF :]
    M = x_hbm.shape[0]
    nsteps = M // _TILE
    w = w_ref[...]
    b = b_ref[...]

    def copy_in(step, slot):
        return pltpu.make_async_copy(
            x_hbm.at[pl.ds(step * _TILE, _TILE), :], xbufs[slot], xsems[slot]
        )

    def copy_out(step, slot):
        return pltpu.make_async_copy(
            obufs[slot], o_hbm.at[pl.ds(step * _TILE, _TILE), :], osems[slot]
        )

    for s in range(_NBUF):
        copy_in(s, s).start()
    for step in range(nsteps):
        slot = step % _NBUF
        copy_in(step, slot).wait()
        if step >= _NBUF:
            copy_out(step - _NBUF, slot).wait()
        obufs[slot][...] = (
            jnp.dot(xbufs[slot][...], w, preferred_element_type=jnp.float32) + b
        )
        copy_out(step, slot).start()
        nxt = step + _NBUF
        if nxt < nsteps:
            copy_in(nxt, slot).start()
    for step in range(nsteps - _NBUF, nsteps):
        copy_out(step, step % _NBUF).wait()


def kernel(local_features, W_coords, b_coords, W_pres, b_pres):
    B, C, R, D = local_features.shape
    M = B * C * R
    x = local_features.reshape(M, D)
    w = jnp.concatenate([W_coords, W_pres], axis=0).T
    b = jnp.concatenate([b_coords, b_pres], axis=0).reshape(1, 5)

    out = pl.pallas_call(
        _fused_heads_kernel,
        in_specs=[
            pl.BlockSpec(memory_space=pl.ANY),
            pl.BlockSpec(memory_space=pltpu.MemorySpace.VMEM),
            pl.BlockSpec(memory_space=pltpu.MemorySpace.VMEM),
        ],
        out_specs=pl.BlockSpec(memory_space=pl.ANY),
        out_shape=jax.ShapeDtypeStruct((M, 5), jnp.float32),
        scratch_shapes=(
            [pltpu.VMEM((_TILE, D), jnp.float32) for _ in range(_NBUF)]
            + [pltpu.VMEM((_TILE, 5), jnp.float32) for _ in range(_NBUF)]
            + [pltpu.SemaphoreType.DMA for _ in range(2 * _NBUF)]
        ),
    )(x, w, b)

    out = out.reshape(B, C, R, 5)
    return (out[..., :4], out[..., 4:])
